# baseline (speedup 1.0000x reference)
import functools

import jax
import jax.numpy as jnp
from jax import lax
from jax.experimental import pallas as pl
from jax.experimental.pallas import tpu as pltpu
from jax.experimental.pallas import tpu_sc as plsc

_THR = 0.999
_N = 16384
_D = 1024
_R = 1024


def _body(a_ref, b_ref, c_ref, wd_ref, out_ref, es_ref):
    a = a_ref[...]
    b = b_ref[...]
    c = c_ref[...]
    wd = wd_ref[...]
    t1 = jnp.sum(a * wd, axis=1, keepdims=True)
    t2 = jnp.sum(b * wd, axis=1, keepdims=True)
    g1 = 1.0 / (1.0 + jnp.exp(-t1))
    g2p = 1.0 / (1.0 + jnp.exp(-t2))
    g2 = jnp.where(g1 >= _THR, 0.0, (1.0 - g1) * g2p)
    acc = g1 + g2
    rem = 1.0 - acc
    coef_c = jnp.where(acc >= _THR, 0.0, rem)
    out_ref[...] = g1 * a + g2 * b + coef_c * c
    es_ref[...] = g2 + 2.0 * rem


_NC = 2
_NS = 16
_NW = _NC * _NS
_RPW = _N // _NW
_RB = 16
_NBLK = _RPW // _RB
_L = 16
_NCH = _D // _L


def _lane_shuffle(x, idx):
    return lax.gather(
        x,
        idx[:, None],
        lax.GatherDimensionNumbers(
            offset_dims=(), collapsed_slice_dims=(0,), start_index_map=(0,)
        ),
        (1,),
        mode=lax.GatherScatterMode.PROMISE_IN_BOUNDS,
    )


def _allsum16(x):
    lanes = lax.iota(jnp.int32, _L)
    for s in (1, 2, 4, 8):
        x = x + _lane_shuffle(x, lanes ^ s)
    return x


def _sc_body(a_hbm, b_hbm, c_hbm, wd_hbm, out_hbm, es_hbm,
             wd_v, a_v, b_v, c_v, o_v, e_v):
    wid = lax.axis_index("s") * _NC + lax.axis_index("c")
    base = wid * _RPW
    pltpu.sync_copy(wd_hbm, wd_v)
    lanes = lax.iota(jnp.int32, _L)

    def blk_body(i, blk_carry):
        row0 = base + i * _RB
        pltpu.sync_copy(a_hbm.at[pl.ds(row0, _RB)], a_v)
        pltpu.sync_copy(b_hbm.at[pl.ds(row0, _RB)], b_v)
        pltpu.sync_copy(c_hbm.at[pl.ds(row0, _RB)], c_v)

        def row_body(r, es_vec):
            acc1 = jnp.zeros((_L,), jnp.float32)
            acc2 = jnp.zeros((_L,), jnp.float32)
            for k in range(_NCH):
                sl = pl.ds(k * _L, _L)
                wdk = wd_v[sl]
                acc1 = acc1 + a_v[r, sl] * wdk
                acc2 = acc2 + b_v[r, sl] * wdk
            tv1 = _allsum16(acc1)
            tv2 = _allsum16(acc2)
            g1 = 1.0 / (1.0 + jnp.exp(-tv1))
            g2p = 1.0 / (1.0 + jnp.exp(-tv2))
            g2 = jnp.where(g1 >= _THR, 0.0, (1.0 - g1) * g2p)
            acc = g1 + g2
            rem = 1.0 - acc
            coef_c = jnp.where(acc >= _THR, 0.0, rem)
            for k in range(_NCH):
                sl = pl.ds(k * _L, _L)
                o_v[r, sl] = g1 * a_v[r, sl] + g2 * b_v[r, sl] + coef_c * c_v[r, sl]
            es_row = g2 + 2.0 * rem
            return jnp.where(lanes == r, es_row, es_vec)

        es_vec = lax.fori_loop(0, _RB, row_body, jnp.zeros((_L,), jnp.float32))
        e_v[...] = es_vec
        pltpu.sync_copy(o_v, out_hbm.at[pl.ds(row0, _RB)])
        pltpu.sync_copy(e_v, es_hbm.at[pl.ds(row0, _RB)])
        return blk_carry

    lax.fori_loop(0, _NBLK, blk_body, 0)


@functools.partial(
    pl.kernel,
    mesh=plsc.VectorSubcoreMesh(core_axis_name="c", subcore_axis_name="s"),
    out_type=[
        jax.ShapeDtypeStruct((_N, _D), jnp.float32),
        jax.ShapeDtypeStruct((_N,), jnp.float32),
    ],
    scratch_types=[
        pltpu.VMEM((_D,), jnp.float32),
        pltpu.VMEM((_RB, _D), jnp.float32),
        pltpu.VMEM((_RB, _D), jnp.float32),
        pltpu.VMEM((_RB, _D), jnp.float32),
        pltpu.VMEM((_RB, _D), jnp.float32),
        pltpu.VMEM((_L,), jnp.float32),
    ],
)
def _sc_kernel(a_hbm, b_hbm, c_hbm, wd_hbm, out_hbm, es_hbm,
               wd_v, a_v, b_v, c_v, o_v, e_v):
    _sc_body(a_hbm, b_hbm, c_hbm, wd_hbm, out_hbm, es_hbm,
             wd_v, a_v, b_v, c_v, o_v, e_v)


@jax.jit
def kernel_sc(prev_out, prev_out2, curr_h, W):
    wd = W[1] - W[0]
    out, es = _sc_kernel(prev_out, prev_out2, curr_h, wd)
    return out, es


@jax.jit
def kernel(prev_out, prev_out2, curr_h, W):
    return kernel_sc(prev_out, prev_out2, curr_h, W)


@jax.jit
def kernel_tc(prev_out, prev_out2, curr_h, W):
    wd = (W[1] - W[0]).reshape(1, _D)
    grid = (_N // _R,)
    row_spec = pl.BlockSpec((_R, _D), lambda i: (i, 0))
    out, es = pl.pallas_call(
        _body,
        grid=grid,
        in_specs=[
            row_spec,
            row_spec,
            row_spec,
            pl.BlockSpec((1, _D), lambda i: (0, 0)),
        ],
        out_specs=[
            row_spec,
            pl.BlockSpec((_R, 1), lambda i: (i, 0)),
        ],
        out_shape=[
            jax.ShapeDtypeStruct((_N, _D), jnp.float32),
            jax.ShapeDtypeStruct((_N, 1), jnp.float32),
        ],
    )(prev_out, prev_out2, curr_h, wd)
    return out, es.reshape(_N)
